# baseline (speedup 1.0000x reference)
import functools

import jax
import jax.numpy as jnp
from jax.experimental import pallas as pl
from jax.experimental.pallas import tpu as pltpu

_NUM_EMB = 512
_EMB_DIM = 256
_COMMIT = 0.25


def _vq_block(z_ref, cb_ref, cbt_ref, csqc_ref, ic_ref, zq_ref, idx_ref, loss_ref):
    t = z_ref.shape[2]
    zb = z_ref[0]
    zsq = jnp.sum(zb * zb, axis=0, keepdims=True)
    dot = jax.lax.dot_general(
        cb_ref[...], zb, (((1,), (0,)), ((), ())),
        preferred_element_type=jnp.float32)
    d = zsq + csqc_ref[...] - 2.0 * dot
    m = jnp.min(d, axis=0, keepdims=True)
    idx = jnp.min(jnp.where(d == m, ic_ref[...], _NUM_EMB),
                  axis=0, keepdims=True)
    onehot = (ic_ref[...] == idx).astype(jnp.bfloat16)
    zq = jax.lax.dot_general(
        cbt_ref[...], onehot, (((1,), (0,)), ((), ())),
        preferred_element_type=jnp.float32)
    zq_ref[0] = zq
    idx_ref[0, 0] = idx[0]
    loss_ref[0] = jnp.sum(m).reshape(1, 1)


@functools.partial(jax.jit, static_argnames=("t_block",))
def _vq(z_e, codebook, t_block=2048):
    b, c, d0, d1, d2 = z_e.shape
    npb = d0 * d1 * d2
    nblk = npb // t_block
    z3 = z_e.reshape(b, c, npb)
    csqc = jnp.sum(codebook ** 2, axis=1, keepdims=True)
    ic = jnp.arange(_NUM_EMB, dtype=jnp.int32).reshape(_NUM_EMB, 1)
    cbt = codebook.T.astype(jnp.bfloat16)

    zq3, idx3, loss = pl.pallas_call(
        _vq_block,
        grid=(b, nblk),
        in_specs=[
            pl.BlockSpec((1, c, t_block), lambda i, j: (i, 0, j)),
            pl.BlockSpec((_NUM_EMB, _EMB_DIM), lambda i, j: (0, 0)),
            pl.BlockSpec((_EMB_DIM, _NUM_EMB), lambda i, j: (0, 0)),
            pl.BlockSpec((_NUM_EMB, 1), lambda i, j: (0, 0)),
            pl.BlockSpec((_NUM_EMB, 1), lambda i, j: (0, 0)),
        ],
        out_specs=[
            pl.BlockSpec((1, c, t_block), lambda i, j: (i, 0, j)),
            pl.BlockSpec((1, 1, t_block), lambda i, j: (i * nblk + j, 0, 0)),
            pl.BlockSpec((1, 1, 1), lambda i, j: (i * nblk + j, 0, 0)),
        ],
        out_shape=[
            jax.ShapeDtypeStruct((b, c, npb), jnp.float32),
            jax.ShapeDtypeStruct((b * nblk, 1, t_block), jnp.int32),
            jax.ShapeDtypeStruct((b * nblk, 1, 1), jnp.float32),
        ],
        compiler_params=pltpu.CompilerParams(
            dimension_semantics=("parallel", "parallel")),
    )(z3, codebook, cbt, csqc, ic)

    z_q = zq3.reshape(b, c, d0, d1, d2)
    indices = idx3.reshape(b, d0, d1, d2)
    vq_loss = jnp.sum(loss) * (1.0 + _COMMIT) / (b * npb * c)
    return z_q, vq_loss, indices


def kernel(z_e, codebook):
    return _vq(z_e, codebook)


# speedup vs baseline: 1.0293x; 1.0293x over previous
import functools

import jax
import jax.numpy as jnp
from jax.experimental import pallas as pl
from jax.experimental.pallas import tpu as pltpu

_NUM_EMB = 512
_EMB_DIM = 256
_COMMIT = 0.25


def _vq_block(z_ref, cb_ref, cbt_ref, csqc_ref, ic_ref, zq_ref, idx_ref, loss_ref):
    zb = z_ref[0]
    zsq = jnp.sum(zb * zb, axis=0, keepdims=True)
    dot = jax.lax.dot_general(
        cb_ref[...], zb, (((1,), (0,)), ((), ())),
        preferred_element_type=jnp.float32)
    d = zsq + csqc_ref[...] - 2.0 * dot
    m = jnp.min(d, axis=0, keepdims=True)
    idx = jnp.min(jnp.where(d == m, ic_ref[...], _NUM_EMB),
                  axis=0, keepdims=True)
    onehot = (ic_ref[...] == idx).astype(jnp.bfloat16)
    zq = jax.lax.dot_general(
        cbt_ref[...], onehot, (((1,), (0,)), ((), ())),
        preferred_element_type=jnp.float32)
    zq_ref[0] = zq
    idx_ref[0, 0] = idx[0]
    loss_ref[0] = jnp.sum(m).reshape(1, 1)


@functools.partial(jax.jit, static_argnames=("t_block",))
def _vq(z_e, codebook, t_block=4096):
    b, c, d0, d1, d2 = z_e.shape
    npb = d0 * d1 * d2
    nblk = npb // t_block
    z3 = z_e.reshape(b, c, npb)
    csqc = jnp.sum(codebook ** 2, axis=1, keepdims=True)
    ic = jnp.arange(_NUM_EMB, dtype=jnp.int32).reshape(_NUM_EMB, 1)
    cbt = codebook.T.astype(jnp.bfloat16)

    zq3, idx3, loss = pl.pallas_call(
        _vq_block,
        grid=(b, nblk),
        in_specs=[
            pl.BlockSpec((1, c, t_block), lambda i, j: (i, 0, j)),
            pl.BlockSpec((_NUM_EMB, _EMB_DIM), lambda i, j: (0, 0)),
            pl.BlockSpec((_EMB_DIM, _NUM_EMB), lambda i, j: (0, 0)),
            pl.BlockSpec((_NUM_EMB, 1), lambda i, j: (0, 0)),
            pl.BlockSpec((_NUM_EMB, 1), lambda i, j: (0, 0)),
        ],
        out_specs=[
            pl.BlockSpec((1, c, t_block), lambda i, j: (i, 0, j)),
            pl.BlockSpec((1, 1, t_block), lambda i, j: (i * nblk + j, 0, 0)),
            pl.BlockSpec((1, 1, 1), lambda i, j: (i * nblk + j, 0, 0)),
        ],
        out_shape=[
            jax.ShapeDtypeStruct((b, c, npb), jnp.float32),
            jax.ShapeDtypeStruct((b * nblk, 1, t_block), jnp.int32),
            jax.ShapeDtypeStruct((b * nblk, 1, 1), jnp.float32),
        ],
        compiler_params=pltpu.CompilerParams(
            dimension_semantics=("parallel", "parallel")),
    )(z3, codebook, cbt, csqc, ic)

    z_q = zq3.reshape(b, c, d0, d1, d2)
    indices = idx3.reshape(b, d0, d1, d2)
    vq_loss = jnp.sum(loss) * (1.0 + _COMMIT) / (b * npb * c)
    return z_q, vq_loss, indices


def kernel(z_e, codebook):
    return _vq(z_e, codebook)
